# baseline (speedup 1.0000x reference)
import jax
import jax.numpy as jnp
from jax import lax
from jax.experimental import pallas as pl
from jax.experimental.pallas import tpu as pltpu
from jax.experimental.pallas import tpu_sc as plsc

_B = 16384
_D = 64
_NC = 2
_NS = 16
_L = 16
_NW = _NC * _NS
_BPW = _B // _NW
_CH = 128
_NCH = _BPW // _CH
_NE2 = 500000
_NR2 = 500


def _sqrt16(x):
    i = plsc.bitcast(x, jnp.int32)
    i = (i >> 1) + jnp.int32(0x1FBD1DF6)
    y = plsc.bitcast(i, jnp.float32)
    for _ in range(3):
        y = 0.5 * (y + x / y)
    return y


def _body(ent_hbm, rel_hbm, hidx_hbm, tidx_hbm, lidx_hbm, out_hbm,
          hidx_v, tidx_v, lidx_v, hp_v, tp_v, lp_v,
          gh0, gt0, gr0, gh1, gt1, gr1, out_v,
          sem0, sem1):
    wid = lax.axis_index("s") * _NC + lax.axis_index("c")
    base = wid * _BPW

    for c in range(_NCH):
        pltpu.sync_copy(hidx_hbm.at[wid * _NCH + c], hidx_v.at[c])
        pltpu.sync_copy(tidx_hbm.at[wid * _NCH + c], tidx_v.at[c])
        pltpu.sync_copy(lidx_hbm.at[wid * _NCH + c], lidx_v.at[c])
    for c in range(_NCH):
        for o in range(_CH // _L):
            sl = pl.ds(o * _L, _L)
            hp_v[c, sl] = hidx_v[c, sl] >> 1
            tp_v[c, sl] = tidx_v[c, sl] >> 1
            lp_v[c, sl] = lidx_v[c, sl] >> 1

    ghs, gts, grs = (gh0, gh1), (gt0, gt1), (gr0, gr1)
    sems = (sem0, sem1)

    ent_copies = {}
    rel_copies = {}
    for b in range(2):
        ent_copies[b] = (
            pltpu.async_copy(ent_hbm.at[hp_v.at[b]], ghs[b], sems[b]),
            pltpu.async_copy(ent_hbm.at[tp_v.at[b]], gts[b], sems[b]),
        )
        rel_copies[b] = pltpu.async_copy(
            rel_hbm.at[lp_v.at[b]], grs[b], sems[b])

    lane = lax.iota(jnp.int32, _L)

    for b in range(_NCH):
        buf = b % 2
        for cp in ent_copies.pop(b):
            cp.wait()
        rel_copies.pop(b).wait()
        gh, gt, gr = ghs[buf], gts[buf], grs[buf]

        @pl.loop(0, _CH // _L)
        def _(g2, b=b, gh=gh, gt=gt, gr=gr):
            sl = pl.ds(g2 * _L, _L)
            qh = (hidx_v[b, sl] & 1) << 6
            qt = (tidx_v[b, sl] & 1) << 6
            qr = (lidx_v[b, sl] & 1) << 6
            rows = lane + g2 * _L
            acc = jnp.zeros((_L,), jnp.float32)
            for d in range(_D):
                off = (lane + d) & (_D - 1)
                vh = plsc.load_gather(gh, [rows, qh + off])
                vt = plsc.load_gather(gt, [rows, qt + off])
                vr = plsc.load_gather(gr, [rows, qr + off])
                s = vh + vr - vt
                acc = acc + s * s
            out_v[pl.ds(b * _CH + g2 * _L, _L)] = _sqrt16(acc)

        nxt = b + 2
        if nxt < _NCH:
            ent_copies[nxt] = (
                pltpu.async_copy(ent_hbm.at[hp_v.at[nxt]], gh, sems[buf]),
                pltpu.async_copy(ent_hbm.at[tp_v.at[nxt]], gt, sems[buf]),
            )
            rel_copies[nxt] = pltpu.async_copy(
                rel_hbm.at[lp_v.at[nxt]], gr, sems[buf])

    pltpu.sync_copy(out_v, out_hbm.at[pl.ds(base, _BPW)])


@jax.jit
def _transe_sc(head, tail, label, entity_emb, relation_emb):
    ent2 = entity_emb.reshape(_NE2, 2 * _D)
    rel2 = relation_emb.reshape(_NR2, 2 * _D)
    hidx = head.astype(jnp.int32).reshape(_NW * _NCH, _CH)
    tidx = tail.astype(jnp.int32).reshape(_NW * _NCH, _CH)
    lidx = label.astype(jnp.int32).reshape(_NW * _NCH, _CH)
    mesh = plsc.VectorSubcoreMesh(core_axis_name="c", subcore_axis_name="s")
    cp = pltpu.CompilerParams(
        needs_layout_passes=False, use_tc_tiling_on_sc=True
    )
    k = pl.kernel(
        _body,
        out_type=jax.ShapeDtypeStruct((_B,), jnp.float32),
        mesh=mesh,
        scratch_types=[
            pltpu.VMEM((_NCH, _CH), jnp.int32),
            pltpu.VMEM((_NCH, _CH), jnp.int32),
            pltpu.VMEM((_NCH, _CH), jnp.int32),
            pltpu.VMEM((_NCH, _CH), jnp.int32),
            pltpu.VMEM((_NCH, _CH), jnp.int32),
            pltpu.VMEM((_NCH, _CH), jnp.int32),
            pltpu.VMEM((_CH, 2 * _D), jnp.float32),
            pltpu.VMEM((_CH, 2 * _D), jnp.float32),
            pltpu.VMEM((_CH, 2 * _D), jnp.float32),
            pltpu.VMEM((_CH, 2 * _D), jnp.float32),
            pltpu.VMEM((_CH, 2 * _D), jnp.float32),
            pltpu.VMEM((_CH, 2 * _D), jnp.float32),
            pltpu.VMEM((_BPW,), jnp.float32),
            pltpu.SemaphoreType.DMA,
            pltpu.SemaphoreType.DMA,
        ],
        compiler_params=cp,
    )
    return k(ent2, rel2, hidx, tidx, lidx)


def kernel(head, tail, label, entity_emb, relation_emb):
    return _transe_sc(head, tail, label, entity_emb, relation_emb)
